# baseline (speedup 1.0000x reference)
import functools

import jax
import jax.numpy as jnp
from jax import lax
from jax.experimental import pallas as pl
from jax.experimental.pallas import tpu as pltpu
from jax.experimental.pallas import tpu_sc as plsc

N = 10000
D = 128
E = 320000
NUM_MASK = 5000

NC, NS = 2, 16
EPS = E // NS
CH = 80
NCHUNK = EPS // CH
NCMAX = NCHUNK + 2
PAD_DST = 10200
NP = 10240
RPS = NP // NS
DEGW = 8
SHIFT = 14
DH = D // 2


def _sc_segment_sum(h2, comb3, maskb, zrow, zdeg, ones_hbm):
    mesh = plsc.VectorSubcoreMesh(core_axis_name="c", subcore_axis_name="s")

    @functools.partial(
        pl.kernel,
        out_type=(
            jax.ShapeDtypeStruct((NC, NP, DH), jnp.float32),
            jax.ShapeDtypeStruct((NC, NP, DEGW), jnp.float32),
        ),
        mesh=mesh,
        scratch_types=[
            pltpu.VMEM((NCHUNK, CH), jnp.int32),
            pltpu.VMEM((NCMAX, CH), jnp.int32),
            pltpu.VMEM((NCMAX, CH), jnp.int32),
            pltpu.VMEM((NP,), jnp.int32),
            pltpu.VMEM((CH, DH), jnp.float32),
            pltpu.VMEM((CH, DH), jnp.float32),
            pltpu.VMEM((CH, DEGW), jnp.float32),
            pltpu.VMEM_SHARED((NP, DH), jnp.float32),
            pltpu.VMEM_SHARED((NP, DEGW), jnp.float32),
            pltpu.SemaphoreType.DMA,
            pltpu.SemaphoreType.DMA,
            pltpu.SemaphoreType.DMA,
            pltpu.SemaphoreType.DMA,
            pltpu.SemaphoreType.DMA,
        ],
        compiler_params=pltpu.CompilerParams(use_tc_tiling_on_sc=False,
                                             needs_layout_passes=False),
    )
    def k(h_hbm, comb_hbm, maskb_hbm, zrow_hbm, zdeg_hbm, ones_h,
          agg_out, deg_out, comb_v, src_v, dst_v, maskb_v, rows_a, rows_b,
          ones_v, agg_s, deg_s, semga, semgb, semsa, semsb, semd):
        cid = lax.axis_index("c")
        sid = lax.axis_index("s")
        row0 = sid * RPS
        for j in range(RPS // CH):
            pltpu.sync_copy(zrow_hbm, agg_s.at[pl.ds(row0 + j * CH, CH)])
            pltpu.sync_copy(zdeg_hbm, deg_s.at[pl.ds(row0 + j * CH, CH)])
        pltpu.sync_copy(comb_hbm.at[sid], comb_v)
        pltpu.sync_copy(maskb_hbm, maskb_v)
        pltpu.sync_copy(ones_h, ones_v)

        mask_lo = (1 << SHIFT) - 1

        def unpack_row(r, cnt):
            for kk in range(CH // 16):
                v = comb_v[r, pl.ds(kk * 16, 16)]
                s2 = lax.shift_left(jnp.bitwise_and(v, mask_lo), 1) + cid
                dst = lax.shift_right_logical(v, SHIFT)
                km_i = plsc.load_gather(maskb_v, [dst])
                km = km_i > 0
                pos = cnt + plsc.cumsum(km_i) - 1
                rowi = pos // CH
                lanei = pos - rowi * CH
                plsc.store_scatter(src_v, [rowi, lanei], s2, mask=km)
                plsc.store_scatter(dst_v, [rowi, lanei], dst, mask=km)
                cnt = cnt + jnp.sum(km_i)
            return cnt

        kcnt = lax.fori_loop(0, NCHUNK, unpack_row, jnp.int32(0))

        nch = 2 * ((kcnt + 2 * CH - 1) // (2 * CH))
        limit = nch * CH
        iota16 = lax.iota(jnp.int32, 16)

        def pad_group(g, cnt):
            pos = kcnt + g * 16 + iota16
            pm = pos < limit
            rowi = pos // CH
            lanei = pos - rowi * CH
            pd = PAD_DST + jnp.bitwise_and(pos, 31)
            plsc.store_scatter(src_v, [rowi, lanei],
                               jnp.zeros((16,), jnp.int32) + cid, mask=pm)
            plsc.store_scatter(dst_v, [rowi, lanei], pd, mask=pm)
            return cnt

        lax.fori_loop(0, 2 * CH // 16, pad_group, jnp.int32(0))
        plsc.subcore_barrier()

        def wait_gather(buf, semx):
            pltpu.make_async_copy(h_hbm.at[src_v.at[0]], buf, semx).wait()

        def wait_scatter(buf, semx):
            pltpu.make_async_copy(buf, agg_s.at[dst_v.at[0]], semx).wait()

        def start_gather(c, buf, semx):
            pltpu.async_copy(h_hbm.at[src_v.at[c]], buf, semx)

        def start_scatter(c, buf, semx):
            pltpu.async_copy(buf, agg_s.at[dst_v.at[c]], semx, add=True)

        npair = nch // 2

        @pl.when(npair > 0)
        def _():
            start_gather(0, rows_a, semga)

            def body(p, carry):
                c0 = 2 * p
                c1 = c0 + 1
                wait_gather(rows_a, semga)

                @pl.when(p > 0)
                def _():
                    wait_scatter(rows_b, semsb)

                start_gather(c1, rows_b, semgb)
                start_scatter(c0, rows_a, semsa)

                @pl.when(cid == 0)
                def _():
                    pltpu.async_copy(ones_v, deg_s.at[dst_v.at[c0]], semd,
                                     add=True)

                wait_gather(rows_b, semgb)
                wait_scatter(rows_a, semsa)

                @pl.when(p < npair - 1)
                def _():
                    start_gather(c0 + 2, rows_a, semga)

                start_scatter(c1, rows_b, semsb)

                @pl.when(cid == 1)
                def _():
                    pltpu.async_copy(ones_v, deg_s.at[dst_v.at[c1]], semd,
                                     add=True)

                return carry

            lax.fori_loop(0, npair, body, 0)
            wait_scatter(rows_b, semsb)

            def drain_deg(i, carry):
                pltpu.make_async_copy(ones_v, deg_s.at[dst_v.at[0]],
                                     semd).wait()
                return carry

            lax.fori_loop(0, npair, drain_deg, 0)

        plsc.subcore_barrier()
        pltpu.sync_copy(agg_s.at[pl.ds(row0, RPS)],
                        agg_out.at[cid, pl.ds(row0, RPS)])
        pltpu.sync_copy(deg_s.at[pl.ds(row0, RPS)],
                        deg_out.at[cid, pl.ds(row0, RPS)])

    return k(h2, comb3, maskb, zrow, zdeg, ones_hbm)


_GRID = 10
_R = NP // _GRID


def _tc_finish(agg2, deg2, x, m2d, W, b2):
    def body(agg_ref, deg_ref, x_ref, m_ref, w_ref, b_ref, out_ref):
        i = pl.program_id(0)
        agg = jnp.concatenate([agg_ref[0], agg_ref[1]], axis=1)
        deg = deg_ref[0, :, 0:1] + deg_ref[1, :, 0:1]
        deg = jnp.maximum(deg, 1.0)
        rec = jnp.dot(agg / deg, w_ref[...],
                      preferred_element_type=jnp.float32) + b_ref[...]
        xv = x_ref[...]
        nr = jnp.sqrt(jnp.sum(rec * rec, axis=1, keepdims=True)) + 1e-8
        nx = jnp.sqrt(jnp.sum(xv * xv, axis=1, keepdims=True)) + 1e-8
        cos = jnp.sum((rec / nr) * (xv / nx), axis=1, keepdims=True)
        t = 1.0 - cos
        part = jnp.sum(t * t * m_ref[:, 0:1])

        @pl.when(i == 0)
        def _():
            out_ref[0, 0] = 0.0

        out_ref[0, 0] += part

    out = pl.pallas_call(
        body,
        grid=(_GRID,),
        in_specs=[
            pl.BlockSpec((NC, _R, DH), lambda i: (0, i, 0)),
            pl.BlockSpec((NC, _R, DEGW), lambda i: (0, i, 0)),
            pl.BlockSpec((_R, D), lambda i: (i, 0)),
            pl.BlockSpec((_R, D), lambda i: (i, 0)),
            pl.BlockSpec((D, D), lambda i: (0, 0)),
            pl.BlockSpec((1, D), lambda i: (0, 0)),
        ],
        out_specs=pl.BlockSpec((1, 1), lambda i: (0, 0),
                               memory_space=pltpu.SMEM),
        out_shape=jax.ShapeDtypeStruct((1, 1), jnp.float32),
    )(agg2, deg2, x, m2d, W, b2)
    return out[0, 0] / float(NUM_MASK)


def kernel(x, h, edge_index, W, b, inference):
    h2 = h.reshape(2 * N, DH)
    comb = jnp.bitwise_or(edge_index[0],
                          jnp.left_shift(edge_index[1], SHIFT))
    comb3 = comb.reshape(NS, NCHUNK, CH)
    zrow = jnp.zeros((CH, DH), jnp.float32)
    zdeg = jnp.zeros((CH, DEGW), jnp.float32)
    ones_hbm = jnp.ones((CH, DEGW), jnp.float32)

    perm = jax.random.permutation(jax.random.key(42), N)
    maskf = jnp.zeros((NP,), jnp.float32).at[perm[:NUM_MASK]].set(1.0)
    maskb = maskf.astype(jnp.int32)
    agg2, deg2 = _sc_segment_sum(h2, comb3, maskb, zrow, zdeg, ones_hbm)
    m2d = jnp.broadcast_to(maskf[:, None], (NP, D))
    xpad = jnp.concatenate([x, jnp.zeros((NP - N, D), jnp.float32)], axis=0)

    loss = _tc_finish(agg2, deg2, xpad, m2d, W, b.reshape(1, D))
    return loss + 0.0 * jnp.asarray(inference, dtype=loss.dtype)


# speedup vs baseline: 1.9465x; 1.9465x over previous
import functools

import jax
import jax.numpy as jnp
from jax import lax
from jax.experimental import pallas as pl
from jax.experimental.pallas import tpu as pltpu
from jax.experimental.pallas import tpu_sc as plsc

N = 10000
D = 128
E = 320000
NUM_MASK = 5000

NC, NS = 2, 16
EPS = E // NS
CH = 80
NCHUNK = 252
EPSP = NCHUNK * CH
PAD_DST = 10000
NP = 10240
RPS = NP // NS
DEGW = 16
SHIFT = 14
DH = D // 2


def _sc_segment_sum(h2, comb3, zrow, zdeg, ones_hbm):
    mesh = plsc.VectorSubcoreMesh(core_axis_name="c", subcore_axis_name="s")

    @functools.partial(
        pl.kernel,
        out_type=(
            jax.ShapeDtypeStruct((NC, NP, DH), jnp.bfloat16),
            jax.ShapeDtypeStruct((NC, NP, DEGW), jnp.float32),
        ),
        mesh=mesh,
        scratch_types=[
            pltpu.VMEM((NCHUNK, CH), jnp.int32),
            pltpu.VMEM((NCHUNK, CH), jnp.int32),
            pltpu.VMEM((NCHUNK, CH), jnp.int32),
            pltpu.VMEM((CH, DH), jnp.bfloat16),
            pltpu.VMEM((CH, DH), jnp.bfloat16),
            pltpu.VMEM((CH, DH), jnp.bfloat16),
            pltpu.VMEM((CH, DH), jnp.bfloat16),
            pltpu.VMEM((CH, DH), jnp.bfloat16),
            pltpu.VMEM((CH, DH), jnp.bfloat16),
            pltpu.VMEM((CH, DEGW), jnp.float32),
            pltpu.VMEM_SHARED((NP, DH), jnp.bfloat16),
            pltpu.VMEM_SHARED((NP, DEGW), jnp.float32),
            pltpu.SemaphoreType.DMA,
            pltpu.SemaphoreType.DMA,
            pltpu.SemaphoreType.DMA,
            pltpu.SemaphoreType.DMA,
            pltpu.SemaphoreType.DMA,
            pltpu.SemaphoreType.DMA,
            pltpu.SemaphoreType.DMA,
            pltpu.SemaphoreType.DMA,
            pltpu.SemaphoreType.DMA,
            pltpu.SemaphoreType.DMA,
            pltpu.SemaphoreType.DMA,
            pltpu.SemaphoreType.DMA,
            pltpu.SemaphoreType.DMA,
        ],
        compiler_params=pltpu.CompilerParams(use_tc_tiling_on_sc=False),
    )
    def k(h_hbm, comb_hbm, zrow_hbm, zdeg_hbm, ones_h,
          agg_out, deg_out, comb_v, src_v, dst_v,
          r0, r1, r2, r3, r4, r5, ones_v, agg_s, deg_s,
          sg0, sg1, sg2, sg3, sg4, sg5,
          ss0, ss1, ss2, ss3, ss4, ss5, semd):
        cid = lax.axis_index("c")
        sid = lax.axis_index("s")
        row0 = sid * RPS
        pltpu.sync_copy(zrow_hbm, agg_s.at[pl.ds(row0, RPS)])
        pltpu.sync_copy(zdeg_hbm, deg_s.at[pl.ds(row0, RPS)])
        pltpu.sync_copy(comb_hbm.at[sid], comb_v)
        pltpu.sync_copy(ones_h, ones_v)

        mask_lo = (1 << SHIFT) - 1

        def unpack_row(r, carry):
            for kk in range(CH // 16):
                v = comb_v[r, pl.ds(kk * 16, 16)]
                s2 = lax.shift_left(jnp.bitwise_and(v, mask_lo), 1) + cid
                src_v[r, pl.ds(kk * 16, 16)] = s2
                dst_v[r, pl.ds(kk * 16, 16)] = lax.shift_right_logical(v, SHIFT)
            return carry

        lax.fori_loop(0, NCHUNK, unpack_row, 0)
        plsc.subcore_barrier()

        def wait_gather(buf, semx):
            pltpu.make_async_copy(h_hbm.at[src_v.at[0]], buf, semx).wait()

        def wait_scatter(buf, semx):
            pltpu.make_async_copy(buf, agg_s.at[dst_v.at[0]], semx).wait()

        def start_gather(c, buf, semx):
            pltpu.async_copy(h_hbm.at[src_v.at[c]], buf, semx)

        def start_scatter(c, buf, semx):
            pltpu.async_copy(buf, agg_s.at[dst_v.at[c]], semx, add=True)

        bufs = (r0, r1, r2, r3, r4, r5)
        gsem = (sg0, sg1, sg2, sg3, sg4, sg5)
        ssem = (ss0, ss1, ss2, ss3, ss4, ss5)
        start_gather(0, r0, sg0)
        start_gather(1, r1, sg1)
        start_gather(2, r2, sg2)

        def body(q, carry):
            for j in range(6):
                c = 6 * q + j
                j3 = (j + 3) % 6

                if j < 3:
                    @pl.when(q > 0)
                    def _():
                        wait_scatter(bufs[j3], ssem[j3])

                    start_gather(c + 3, bufs[j3], gsem[j3])
                else:
                    wait_scatter(bufs[j3], ssem[j3])

                    @pl.when(q < NCHUNK // 6 - 1)
                    def _():
                        start_gather(c + 3, bufs[j3], gsem[j3])

                wait_gather(bufs[j], gsem[j])
                start_scatter(c, bufs[j], ssem[j])

                @pl.when(cid == c % 2)
                def _():
                    pltpu.async_copy(ones_v, deg_s.at[dst_v.at[c]], semd,
                                     add=True)

            return carry

        lax.fori_loop(0, NCHUNK // 6, body, 0)
        wait_scatter(r3, ss3)
        wait_scatter(r4, ss4)
        wait_scatter(r5, ss5)

        def drain_deg(i, carry):
            pltpu.make_async_copy(ones_v, deg_s.at[dst_v.at[0]], semd).wait()
            return carry

        lax.fori_loop(0, NCHUNK // 2, drain_deg, 0)
        plsc.subcore_barrier()
        pltpu.sync_copy(agg_s.at[pl.ds(row0, RPS)],
                        agg_out.at[cid, pl.ds(row0, RPS)])
        pltpu.sync_copy(deg_s.at[pl.ds(row0, RPS)],
                        deg_out.at[cid, pl.ds(row0, RPS)])

    return k(h2, comb3, zrow, zdeg, ones_hbm)


_GRID = 10
_R = NP // _GRID


def _tc_finish(agg2, deg2, x, m2d, W, b2):
    def body(agg_ref, deg_ref, x_ref, m_ref, w_ref, b_ref, out_ref):
        i = pl.program_id(0)
        agg = jnp.concatenate([agg_ref[0], agg_ref[1]],
                              axis=1).astype(jnp.float32)
        deg = deg_ref[0, :, 0:1] + deg_ref[1, :, 0:1]
        deg = jnp.maximum(deg, 1.0)
        rec = jnp.dot(agg / deg, w_ref[...],
                      preferred_element_type=jnp.float32) + b_ref[...]
        xv = x_ref[...]
        nr = jnp.sqrt(jnp.sum(rec * rec, axis=1, keepdims=True)) + 1e-8
        nx = jnp.sqrt(jnp.sum(xv * xv, axis=1, keepdims=True)) + 1e-8
        cos = jnp.sum((rec / nr) * (xv / nx), axis=1, keepdims=True)
        t = 1.0 - cos
        part = jnp.sum(t * t * m_ref[:, 0:1])

        @pl.when(i == 0)
        def _():
            out_ref[0, 0] = 0.0

        out_ref[0, 0] += part

    out = pl.pallas_call(
        body,
        grid=(_GRID,),
        in_specs=[
            pl.BlockSpec((NC, _R, DH), lambda i: (0, i, 0)),
            pl.BlockSpec((NC, _R, DEGW), lambda i: (0, i, 0)),
            pl.BlockSpec((_R, D), lambda i: (i, 0)),
            pl.BlockSpec((_R, D), lambda i: (i, 0)),
            pl.BlockSpec((D, D), lambda i: (0, 0)),
            pl.BlockSpec((1, D), lambda i: (0, 0)),
        ],
        out_specs=pl.BlockSpec((1, 1), lambda i: (0, 0),
                               memory_space=pltpu.SMEM),
        out_shape=jax.ShapeDtypeStruct((1, 1), jnp.float32),
    )(agg2, deg2, x, m2d, W, b2)
    return out[0, 0] / float(NUM_MASK)


def kernel(x, h, edge_index, W, b, inference):
    h2 = h.reshape(2 * N, DH).astype(jnp.bfloat16)
    comb = jnp.bitwise_or(edge_index[0],
                          jnp.left_shift(edge_index[1], SHIFT))
    padrow = (PAD_DST + jnp.arange(EPSP - EPS, dtype=jnp.int32)) << SHIFT
    comb3 = jnp.concatenate(
        [comb.reshape(NS, EPS),
         jnp.broadcast_to(padrow, (NS, EPSP - EPS))], axis=1
    ).reshape(NS, NCHUNK, CH)
    zrow = jnp.zeros((RPS, DH), jnp.bfloat16)
    zdeg = jnp.zeros((RPS, DEGW), jnp.float32)
    ones_hbm = jnp.ones((CH, DEGW), jnp.float32)
    agg2, deg2 = _sc_segment_sum(h2, comb3, zrow, zdeg, ones_hbm)

    perm = jax.random.permutation(jax.random.key(42), N)
    maskf = jnp.zeros((NP,), jnp.float32).at[perm[:NUM_MASK]].set(1.0)
    m2d = jnp.broadcast_to(maskf[:, None], (NP, D))
    xpad = jnp.concatenate([x, jnp.zeros((NP - N, D), jnp.float32)], axis=0)

    loss = _tc_finish(agg2, deg2, xpad, m2d, W, b.reshape(1, D))
    return loss + 0.0 * jnp.asarray(inference, dtype=loss.dtype)


# speedup vs baseline: 1.9874x; 1.0210x over previous
import functools

import jax
import jax.numpy as jnp
from jax import lax
from jax.experimental import pallas as pl
from jax.experimental.pallas import tpu as pltpu
from jax.experimental.pallas import tpu_sc as plsc

N = 10000
D = 128
E = 320000
NUM_MASK = 5000

NC, NS = 2, 16
EPS = E // NS
CH = 96
NCHUNK = 210
EPSP = NCHUNK * CH
PAD_DST = 10000
NP = 10240
RPS = NP // NS
DEGW = 16
SHIFT = 14
DH = D // 2


def _sc_segment_sum(h2, comb3, zrow, zdeg, ones_hbm):
    mesh = plsc.VectorSubcoreMesh(core_axis_name="c", subcore_axis_name="s")

    @functools.partial(
        pl.kernel,
        out_type=(
            jax.ShapeDtypeStruct((NC, NP, DH), jnp.bfloat16),
            jax.ShapeDtypeStruct((NC, NP, DEGW), jnp.float32),
        ),
        mesh=mesh,
        scratch_types=[
            pltpu.VMEM((NCHUNK, CH), jnp.int32),
            pltpu.VMEM((NCHUNK, CH), jnp.int32),
            pltpu.VMEM((NCHUNK, CH), jnp.int32),
            pltpu.VMEM((CH, DH), jnp.bfloat16),
            pltpu.VMEM((CH, DH), jnp.bfloat16),
            pltpu.VMEM((CH, DH), jnp.bfloat16),
            pltpu.VMEM((CH, DH), jnp.bfloat16),
            pltpu.VMEM((CH, DH), jnp.bfloat16),
            pltpu.VMEM((CH, DH), jnp.bfloat16),
            pltpu.VMEM((CH, DEGW), jnp.float32),
            pltpu.VMEM_SHARED((NP, DH), jnp.bfloat16),
            pltpu.VMEM_SHARED((NP, DEGW), jnp.float32),
            pltpu.SemaphoreType.DMA,
            pltpu.SemaphoreType.DMA,
            pltpu.SemaphoreType.DMA,
            pltpu.SemaphoreType.DMA,
            pltpu.SemaphoreType.DMA,
            pltpu.SemaphoreType.DMA,
            pltpu.SemaphoreType.DMA,
            pltpu.SemaphoreType.DMA,
            pltpu.SemaphoreType.DMA,
            pltpu.SemaphoreType.DMA,
            pltpu.SemaphoreType.DMA,
            pltpu.SemaphoreType.DMA,
            pltpu.SemaphoreType.DMA,
        ],
        compiler_params=pltpu.CompilerParams(use_tc_tiling_on_sc=False),
    )
    def k(h_hbm, comb_hbm, zrow_hbm, zdeg_hbm, ones_h,
          agg_out, deg_out, comb_v, src_v, dst_v,
          r0, r1, r2, r3, r4, r5, ones_v, agg_s, deg_s,
          sg0, sg1, sg2, sg3, sg4, sg5,
          ss0, ss1, ss2, ss3, ss4, ss5, semd):
        cid = lax.axis_index("c")
        sid = lax.axis_index("s")
        row0 = sid * RPS
        pltpu.sync_copy(zrow_hbm, agg_s.at[pl.ds(row0, RPS)])
        pltpu.sync_copy(zdeg_hbm, deg_s.at[pl.ds(row0, RPS)])
        pltpu.sync_copy(comb_hbm.at[sid], comb_v)
        pltpu.sync_copy(ones_h, ones_v)

        mask_lo = (1 << SHIFT) - 1

        def unpack_row(r, carry):
            for kk in range(CH // 16):
                v = comb_v[r, pl.ds(kk * 16, 16)]
                s2 = lax.shift_left(jnp.bitwise_and(v, mask_lo), 1) + cid
                src_v[r, pl.ds(kk * 16, 16)] = s2
                dst_v[r, pl.ds(kk * 16, 16)] = lax.shift_right_logical(v, SHIFT)
            return carry

        lax.fori_loop(0, NCHUNK, unpack_row, 0)
        plsc.subcore_barrier()

        def wait_gather(buf, semx):
            pltpu.make_async_copy(h_hbm.at[src_v.at[0]], buf, semx).wait()

        def wait_scatter(buf, semx):
            pltpu.make_async_copy(buf, agg_s.at[dst_v.at[0]], semx).wait()

        def start_gather(c, buf, semx):
            pltpu.async_copy(h_hbm.at[src_v.at[c]], buf, semx)

        def start_scatter(c, buf, semx):
            pltpu.async_copy(buf, agg_s.at[dst_v.at[c]], semx, add=True)

        bufs = (r0, r1, r2, r3, r4, r5)
        gsem = (sg0, sg1, sg2, sg3, sg4, sg5)
        ssem = (ss0, ss1, ss2, ss3, ss4, ss5)
        start_gather(0, r0, sg0)
        start_gather(1, r1, sg1)
        start_gather(2, r2, sg2)

        def body(q, carry):
            for j in range(6):
                c = 6 * q + j
                j3 = (j + 3) % 6

                if j < 3:
                    @pl.when(q > 0)
                    def _():
                        wait_scatter(bufs[j3], ssem[j3])

                    start_gather(c + 3, bufs[j3], gsem[j3])
                else:
                    wait_scatter(bufs[j3], ssem[j3])

                    @pl.when(q < NCHUNK // 6 - 1)
                    def _():
                        start_gather(c + 3, bufs[j3], gsem[j3])

                wait_gather(bufs[j], gsem[j])
                start_scatter(c, bufs[j], ssem[j])

                @pl.when(cid == c % 2)
                def _():
                    pltpu.async_copy(ones_v, deg_s.at[dst_v.at[c]], semd,
                                     add=True)

            return carry

        lax.fori_loop(0, NCHUNK // 6, body, 0)
        wait_scatter(r3, ss3)
        wait_scatter(r4, ss4)
        wait_scatter(r5, ss5)

        def drain_deg(i, carry):
            pltpu.make_async_copy(ones_v, deg_s.at[dst_v.at[0]], semd).wait()
            return carry

        lax.fori_loop(0, NCHUNK // 2, drain_deg, 0)
        plsc.subcore_barrier()
        pltpu.sync_copy(agg_s.at[pl.ds(row0, RPS)],
                        agg_out.at[cid, pl.ds(row0, RPS)])
        pltpu.sync_copy(deg_s.at[pl.ds(row0, RPS)],
                        deg_out.at[cid, pl.ds(row0, RPS)])

    return k(h2, comb3, zrow, zdeg, ones_hbm)


_GRID = 10
_R = NP // _GRID


def _tc_finish(agg2, deg2, x, m2d, W, b2):
    def body(agg_ref, deg_ref, x_ref, m_ref, w_ref, b_ref, out_ref):
        i = pl.program_id(0)
        agg = jnp.concatenate([agg_ref[0], agg_ref[1]],
                              axis=1).astype(jnp.float32)
        deg = deg_ref[0, :, 0:1] + deg_ref[1, :, 0:1]
        deg = jnp.maximum(deg, 1.0)
        rec = jnp.dot(agg / deg, w_ref[...],
                      preferred_element_type=jnp.float32) + b_ref[...]
        xv = x_ref[...]
        nr = jnp.sqrt(jnp.sum(rec * rec, axis=1, keepdims=True)) + 1e-8
        nx = jnp.sqrt(jnp.sum(xv * xv, axis=1, keepdims=True)) + 1e-8
        cos = jnp.sum((rec / nr) * (xv / nx), axis=1, keepdims=True)
        t = 1.0 - cos
        part = jnp.sum(t * t * m_ref[:, 0:1])

        @pl.when(i == 0)
        def _():
            out_ref[0, 0] = 0.0

        out_ref[0, 0] += part

    out = pl.pallas_call(
        body,
        grid=(_GRID,),
        in_specs=[
            pl.BlockSpec((NC, _R, DH), lambda i: (0, i, 0)),
            pl.BlockSpec((NC, _R, DEGW), lambda i: (0, i, 0)),
            pl.BlockSpec((_R, D), lambda i: (i, 0)),
            pl.BlockSpec((_R, D), lambda i: (i, 0)),
            pl.BlockSpec((D, D), lambda i: (0, 0)),
            pl.BlockSpec((1, D), lambda i: (0, 0)),
        ],
        out_specs=pl.BlockSpec((1, 1), lambda i: (0, 0),
                               memory_space=pltpu.SMEM),
        out_shape=jax.ShapeDtypeStruct((1, 1), jnp.float32),
    )(agg2, deg2, x, m2d, W, b2)
    return out[0, 0] / float(NUM_MASK)


def kernel(x, h, edge_index, W, b, inference):
    h2 = h.reshape(2 * N, DH).astype(jnp.bfloat16)
    comb = jnp.bitwise_or(edge_index[0],
                          jnp.left_shift(edge_index[1], SHIFT))
    padrow = (PAD_DST + jnp.arange(EPSP - EPS, dtype=jnp.int32)) << SHIFT
    comb3 = jnp.concatenate(
        [comb.reshape(NS, EPS),
         jnp.broadcast_to(padrow, (NS, EPSP - EPS))], axis=1
    ).reshape(NS, NCHUNK, CH)
    zrow = jnp.zeros((RPS, DH), jnp.bfloat16)
    zdeg = jnp.zeros((RPS, DEGW), jnp.float32)
    ones_hbm = jnp.ones((CH, DEGW), jnp.float32)
    agg2, deg2 = _sc_segment_sum(h2, comb3, zrow, zdeg, ones_hbm)

    perm = jax.random.permutation(jax.random.key(42), N)
    maskf = jnp.zeros((NP,), jnp.float32).at[perm[:NUM_MASK]].set(1.0)
    m2d = jnp.broadcast_to(maskf[:, None], (NP, D))
    xpad = jnp.concatenate([x, jnp.zeros((NP - N, D), jnp.float32)], axis=0)

    loss = _tc_finish(agg2, deg2, xpad, m2d, W, b.reshape(1, D))
    return loss + 0.0 * jnp.asarray(inference, dtype=loss.dtype)


# speedup vs baseline: 2.0022x; 1.0074x over previous
import functools

import jax
import jax.numpy as jnp
from jax import lax
from jax.experimental import pallas as pl
from jax.experimental.pallas import tpu as pltpu
from jax.experimental.pallas import tpu_sc as plsc

N = 10000
D = 128
E = 320000
NUM_MASK = 5000

NC, NS = 2, 16
EPS = E // NS
CH = 112
NCHUNK = 180
EPSP = NCHUNK * CH
PAD_DST = 10000
NP = 10240
RPS = NP // NS
DEGW = 16
SHIFT = 14
DH = D // 2


def _sc_segment_sum(h2, comb3, zrow, zdeg, ones_hbm):
    mesh = plsc.VectorSubcoreMesh(core_axis_name="c", subcore_axis_name="s")

    @functools.partial(
        pl.kernel,
        out_type=(
            jax.ShapeDtypeStruct((NC, NP, DH), jnp.bfloat16),
            jax.ShapeDtypeStruct((NC, NP, DEGW), jnp.float32),
        ),
        mesh=mesh,
        scratch_types=[
            pltpu.VMEM((NCHUNK, CH), jnp.int32),
            pltpu.VMEM((NCHUNK, CH), jnp.int32),
            pltpu.VMEM((NCHUNK, CH), jnp.int32),
            pltpu.VMEM((CH, DH), jnp.bfloat16),
            pltpu.VMEM((CH, DH), jnp.bfloat16),
            pltpu.VMEM((CH, DH), jnp.bfloat16),
            pltpu.VMEM((CH, DH), jnp.bfloat16),
            pltpu.VMEM((CH, DH), jnp.bfloat16),
            pltpu.VMEM((CH, DH), jnp.bfloat16),
            pltpu.VMEM((CH, DEGW), jnp.float32),
            pltpu.VMEM_SHARED((NP, DH), jnp.bfloat16),
            pltpu.VMEM_SHARED((NP, DEGW), jnp.float32),
            pltpu.SemaphoreType.DMA,
            pltpu.SemaphoreType.DMA,
            pltpu.SemaphoreType.DMA,
            pltpu.SemaphoreType.DMA,
            pltpu.SemaphoreType.DMA,
            pltpu.SemaphoreType.DMA,
            pltpu.SemaphoreType.DMA,
            pltpu.SemaphoreType.DMA,
            pltpu.SemaphoreType.DMA,
            pltpu.SemaphoreType.DMA,
            pltpu.SemaphoreType.DMA,
            pltpu.SemaphoreType.DMA,
            pltpu.SemaphoreType.DMA,
        ],
        compiler_params=pltpu.CompilerParams(use_tc_tiling_on_sc=False),
    )
    def k(h_hbm, comb_hbm, zrow_hbm, zdeg_hbm, ones_h,
          agg_out, deg_out, comb_v, src_v, dst_v,
          r0, r1, r2, r3, r4, r5, ones_v, agg_s, deg_s,
          sg0, sg1, sg2, sg3, sg4, sg5,
          ss0, ss1, ss2, ss3, ss4, ss5, semd):
        cid = lax.axis_index("c")
        sid = lax.axis_index("s")
        row0 = sid * RPS
        pltpu.sync_copy(zrow_hbm, agg_s.at[pl.ds(row0, RPS)])
        pltpu.sync_copy(zdeg_hbm, deg_s.at[pl.ds(row0, RPS)])
        pltpu.sync_copy(comb_hbm.at[sid], comb_v)
        pltpu.sync_copy(ones_h, ones_v)

        mask_lo = (1 << SHIFT) - 1

        def unpack_row(r, carry):
            for kk in range(CH // 16):
                v = comb_v[r, pl.ds(kk * 16, 16)]
                s2 = lax.shift_left(jnp.bitwise_and(v, mask_lo), 1) + cid
                src_v[r, pl.ds(kk * 16, 16)] = s2
                dst_v[r, pl.ds(kk * 16, 16)] = lax.shift_right_logical(v, SHIFT)
            return carry

        lax.fori_loop(0, NCHUNK, unpack_row, 0)
        plsc.subcore_barrier()

        def wait_gather(buf, semx):
            pltpu.make_async_copy(h_hbm.at[src_v.at[0]], buf, semx).wait()

        def wait_scatter(buf, semx):
            pltpu.make_async_copy(buf, agg_s.at[dst_v.at[0]], semx).wait()

        def start_gather(c, buf, semx):
            pltpu.async_copy(h_hbm.at[src_v.at[c]], buf, semx)

        def start_scatter(c, buf, semx):
            pltpu.async_copy(buf, agg_s.at[dst_v.at[c]], semx, add=True)

        bufs = (r0, r1, r2, r3, r4, r5)
        gsem = (sg0, sg1, sg2, sg3, sg4, sg5)
        ssem = (ss0, ss1, ss2, ss3, ss4, ss5)
        start_gather(0, r0, sg0)
        start_gather(1, r1, sg1)
        start_gather(2, r2, sg2)

        def body(q, carry):
            for j in range(6):
                c = 6 * q + j
                j3 = (j + 3) % 6

                if j < 3:
                    @pl.when(q > 0)
                    def _():
                        wait_scatter(bufs[j3], ssem[j3])

                    start_gather(c + 3, bufs[j3], gsem[j3])
                else:
                    wait_scatter(bufs[j3], ssem[j3])

                    @pl.when(q < NCHUNK // 6 - 1)
                    def _():
                        start_gather(c + 3, bufs[j3], gsem[j3])

                wait_gather(bufs[j], gsem[j])
                start_scatter(c, bufs[j], ssem[j])

                @pl.when(cid == c % 2)
                def _():
                    pltpu.async_copy(ones_v, deg_s.at[dst_v.at[c]], semd,
                                     add=True)

            return carry

        lax.fori_loop(0, NCHUNK // 6, body, 0)
        wait_scatter(r3, ss3)
        wait_scatter(r4, ss4)
        wait_scatter(r5, ss5)

        def drain_deg(i, carry):
            pltpu.make_async_copy(ones_v, deg_s.at[dst_v.at[0]], semd).wait()
            return carry

        lax.fori_loop(0, NCHUNK // 2, drain_deg, 0)
        plsc.subcore_barrier()
        pltpu.sync_copy(agg_s.at[pl.ds(row0, RPS)],
                        agg_out.at[cid, pl.ds(row0, RPS)])
        pltpu.sync_copy(deg_s.at[pl.ds(row0, RPS)],
                        deg_out.at[cid, pl.ds(row0, RPS)])

    return k(h2, comb3, zrow, zdeg, ones_hbm)


_GRID = 10
_R = NP // _GRID


def _tc_finish(agg2, deg2, x, m2d, W, b2):
    def body(agg_ref, deg_ref, x_ref, m_ref, w_ref, b_ref, out_ref):
        i = pl.program_id(0)
        agg = jnp.concatenate([agg_ref[0], agg_ref[1]],
                              axis=1).astype(jnp.float32)
        deg = deg_ref[0, :, 0:1] + deg_ref[1, :, 0:1]
        deg = jnp.maximum(deg, 1.0)
        rec = jnp.dot(agg / deg, w_ref[...],
                      preferred_element_type=jnp.float32) + b_ref[...]
        xv = x_ref[...]
        nr = jnp.sqrt(jnp.sum(rec * rec, axis=1, keepdims=True)) + 1e-8
        nx = jnp.sqrt(jnp.sum(xv * xv, axis=1, keepdims=True)) + 1e-8
        cos = jnp.sum((rec / nr) * (xv / nx), axis=1, keepdims=True)
        t = 1.0 - cos
        part = jnp.sum(t * t * m_ref[:, 0:1])

        @pl.when(i == 0)
        def _():
            out_ref[0, 0] = 0.0

        out_ref[0, 0] += part

    out = pl.pallas_call(
        body,
        grid=(_GRID,),
        in_specs=[
            pl.BlockSpec((NC, _R, DH), lambda i: (0, i, 0)),
            pl.BlockSpec((NC, _R, DEGW), lambda i: (0, i, 0)),
            pl.BlockSpec((_R, D), lambda i: (i, 0)),
            pl.BlockSpec((_R, D), lambda i: (i, 0)),
            pl.BlockSpec((D, D), lambda i: (0, 0)),
            pl.BlockSpec((1, D), lambda i: (0, 0)),
        ],
        out_specs=pl.BlockSpec((1, 1), lambda i: (0, 0),
                               memory_space=pltpu.SMEM),
        out_shape=jax.ShapeDtypeStruct((1, 1), jnp.float32),
    )(agg2, deg2, x, m2d, W, b2)
    return out[0, 0] / float(NUM_MASK)


def kernel(x, h, edge_index, W, b, inference):
    h2 = h.reshape(2 * N, DH).astype(jnp.bfloat16)
    comb = jnp.bitwise_or(edge_index[0],
                          jnp.left_shift(edge_index[1], SHIFT))
    padrow = (PAD_DST + jnp.arange(EPSP - EPS, dtype=jnp.int32)) << SHIFT
    comb3 = jnp.concatenate(
        [comb.reshape(NS, EPS),
         jnp.broadcast_to(padrow, (NS, EPSP - EPS))], axis=1
    ).reshape(NS, NCHUNK, CH)
    zrow = jnp.zeros((RPS, DH), jnp.bfloat16)
    zdeg = jnp.zeros((RPS, DEGW), jnp.float32)
    ones_hbm = jnp.ones((CH, DEGW), jnp.float32)
    agg2, deg2 = _sc_segment_sum(h2, comb3, zrow, zdeg, ones_hbm)

    perm = jax.random.permutation(jax.random.key(42), N)
    maskf = jnp.zeros((NP,), jnp.float32).at[perm[:NUM_MASK]].set(1.0)
    m2d = jnp.broadcast_to(maskf[:, None], (NP, D))
    xpad = jnp.concatenate([x, jnp.zeros((NP - N, D), jnp.float32)], axis=0)

    loss = _tc_finish(agg2, deg2, xpad, m2d, W, b.reshape(1, D))
    return loss + 0.0 * jnp.asarray(inference, dtype=loss.dtype)
